# baseline (speedup 1.0000x reference)
import jax
import jax.numpy as jnp
from jax import lax
from jax.experimental import pallas as pl
from jax.experimental.pallas import tpu as pltpu

R = 16384
C = 1000
K = int(R * 0.7)
BR = 4096
NB = R // BR


def _ohem_kernel(predt_ref, tgt_ref, out_ref, loss_sc):
    i = pl.program_id(0)
    x = predt_ref[...]
    m = jnp.max(x, axis=0)
    e = jnp.exp(x - m[None, :])
    s = lax.dot_general(
        jnp.ones((1, C), jnp.float32), e,
        (((1,), (0,)), ((), ())),
        preferred_element_type=jnp.float32,
    )[0]
    lse = m + jnp.log(s)
    tgt = tgt_ref[0, 0, :]
    row = lax.broadcasted_iota(jnp.int32, (C, BR), 0)
    picked = jnp.sum(jnp.where(row == tgt[None, :], x, 0.0), axis=0)
    loss_sc[i, :] = lse - picked

    @pl.when(i == NB - 1)
    def _():
        vals = loss_sc[...]
        u = lax.bitcast_convert_type(vals, jnp.uint32)
        sk = u ^ jnp.where(
            u >= jnp.uint32(0x80000000),
            jnp.uint32(0xFFFFFFFF),
            jnp.uint32(0x80000000),
        )

        def body(it, p):
            cand = p | (jnp.uint32(1) << (31 - it).astype(jnp.uint32))
            cnt = jnp.sum((sk >= cand).astype(jnp.int32))
            return jnp.where(cnt >= K, cand, p)

        p = lax.fori_loop(0, 32, body, jnp.uint32(0))

        gt = sk > p
        cnt_gt = jnp.sum(gt.astype(jnp.int32))
        sum_gt = jnp.sum(jnp.where(gt, vals, 0.0))
        orig = jnp.where(
            (p & jnp.uint32(0x80000000)) != jnp.uint32(0),
            p ^ jnp.uint32(0x80000000),
            ~p,
        )
        tau = lax.bitcast_convert_type(orig, jnp.float32)
        total = sum_gt + (K - cnt_gt).astype(jnp.float32) * tau
        out_ref[0, 0] = total / K


def kernel(pred, target):
    predt = pred.T
    tgt = target.astype(jnp.int32).reshape(NB, 1, BR)
    out = pl.pallas_call(
        _ohem_kernel,
        grid=(NB,),
        in_specs=[
            pl.BlockSpec((C, BR), lambda i: (0, i)),
            pl.BlockSpec((1, 1, BR), lambda i: (i, 0, 0)),
        ],
        out_specs=pl.BlockSpec(
            (1, 1), lambda i: (0, 0), memory_space=pltpu.SMEM
        ),
        out_shape=jax.ShapeDtypeStruct((1, 1), jnp.float32),
        scratch_shapes=[pltpu.VMEM((NB, BR), jnp.float32)],
    )(predt, tgt)
    return out[0, 0]
